# baseline (speedup 1.0000x reference)
import functools

import jax
import jax.numpy as jnp
from jax import lax
from jax.experimental import pallas as pl
from jax.experimental.pallas import tpu as pltpu
from jax.experimental.pallas import tpu_sc as plsc

_VOCAB = 1000000
_EMB = 64
_HID = 128
_B = 16384
_L = 50

_NC, _NS, _LANES = 2, 16, 16
_NW = _NC * _NS


def _fold_body(w1_ref, b1_ref, w2_ref, b2_ref, wc_ref):
    w2 = w2_ref[...]
    w = jnp.dot(w2, w1_ref[...], preferred_element_type=jnp.float32,
                precision=lax.Precision.HIGHEST)
    c = jnp.sum(w2 * b1_ref[...]) + b2_ref[0, 0]
    wc_ref[:, :_EMB] = w
    wc_ref[:, _EMB:] = jnp.full((1, 16), c, jnp.float32)


_fold = pl.pallas_call(
    _fold_body,
    out_shape=jax.ShapeDtypeStruct((1, _EMB + 16), jnp.float32),
)


_CR = 400
_NCHT = _VOCAB // _CR
_JMAX = 80
_NP = _EMB // _LANES

_sc_mesh = plsc.VectorSubcoreMesh(core_axis_name="c", subcore_axis_name="s")


@functools.partial(
    pl.kernel,
    mesh=_sc_mesh,
    out_type=jax.ShapeDtypeStruct((_VOCAB,), jnp.float32),
    compiler_params=pltpu.CompilerParams(needs_layout_passes=False),
    scratch_types=[
        pltpu.VMEM((2, _CR, _EMB), jnp.float32),
        pltpu.VMEM((1, _EMB + 16), jnp.float32),
        pltpu.VMEM((_CR,), jnp.float32),
        pltpu.SemaphoreType.DMA,
        pltpu.SemaphoreType.DMA,
    ],
)
def _score(table_hbm, wc_hbm, s_hbm, rows_v, wc_v, sv, sem0, sem1):
    wid = lax.axis_index("s") * _NC + lax.axis_index("c")
    pltpu.sync_copy(wc_hbm, wc_v)
    w_p = [wc_v[0, pl.ds(p * _LANES, _LANES)] for p in range(_NP)]
    cvec = wc_v[0, pl.ds(_EMB, _LANES)]
    lane = jnp.arange(_LANES, dtype=jnp.int32)
    sems = (sem0, sem1)

    def fire(j, buf):
        cid = wid + j * _NW

        @pl.when(cid < _NCHT)
        def _():
            pltpu.async_copy(
                table_hbm.at[pl.ds(cid * _CR, _CR), :], rows_v.at[buf],
                sems[buf])

    def process(j, buf):
        cid = wid + j * _NW

        @pl.when(cid < _NCHT)
        def _():
            pltpu.make_async_copy(
                table_hbm.at[pl.ds(cid * _CR, _CR), :], rows_v.at[buf],
                sems[buf]).wait()

            def row_body(r, carry):
                part = rows_v[buf, r, pl.ds(0, _LANES)] * w_p[0]
                for p in range(1, _NP):
                    part = part + rows_v[buf, r,
                                         pl.ds(p * _LANES, _LANES)] * w_p[p]
                tot = jnp.sum(part)
                plsc.store_scatter(
                    sv,
                    [jnp.full((_LANES,), r, jnp.int32)],
                    jnp.full((_LANES,), tot, jnp.float32),
                    mask=lane == 0,
                )
                return carry

            lax.fori_loop(0, _CR, row_body, 0)

            def post(g, carry):
                sl = pl.ds(g * _LANES, _LANES)
                sv[sl] = (sv[sl] + cvec) * (1.0 / _L)
                return carry

            lax.fori_loop(0, _CR // _LANES, post, 0)
            pltpu.sync_copy(sv, s_hbm.at[pl.ds(cid * _CR, _CR)])

    fire(0, 0)

    def outer(i2, carry):
        j0 = i2 * 2
        fire(j0 + 1, 1)
        process(j0, 0)
        fire(j0 + 2, 0)
        process(j0 + 1, 1)
        return carry

    lax.fori_loop(0, _JMAX // 2, outer, 0)


_TOK_W = (_B * _L) // _NW
_BAG_W = _B // _NW
_GRP = _BAG_W // _LANES


@functools.partial(
    pl.kernel,
    mesh=_sc_mesh,
    out_type=jax.ShapeDtypeStruct((_B,), jnp.float32),
    compiler_params=pltpu.CompilerParams(needs_layout_passes=False),
    scratch_types=[
        pltpu.VMEM((_TOK_W,), jnp.int32),
        pltpu.VMEM((_TOK_W,), jnp.float32),
        pltpu.VMEM((_BAG_W,), jnp.float32),
        pltpu.SemaphoreType.DMA,
    ],
)
def _pool(s_hbm, idx_hbm, out_hbm, idx_v, val_v, out_v, sem):
    wid = lax.axis_index("s") * _NC + lax.axis_index("c")
    tbase = wid * _TOK_W
    pltpu.sync_copy(idx_hbm.at[pl.ds(tbase, _TOK_W)], idx_v)
    pltpu.async_copy(s_hbm.at[idx_v], val_v, sem).wait()
    lane50 = jnp.arange(_LANES, dtype=jnp.int32) * _L

    def body(g, carry):
        base = lane50 + g * (_LANES * _L)
        acc = plsc.load_gather(val_v, [base])
        for l in range(1, _L):
            acc = acc + plsc.load_gather(val_v, [base + l])
        out_v[pl.ds(g * _LANES, _LANES)] = acc
        return carry

    lax.fori_loop(0, _GRP, body, 0)
    pltpu.sync_copy(out_v, out_hbm.at[pl.ds(wid * _BAG_W, _BAG_W)])


def kernel(input_batch, emb_table, W1, b1, W2, b2):
    wc = _fold(W1, b1.reshape(1, _HID), W2, b2.reshape(1, 1))
    s = _score(emb_table, wc)
    out = _pool(s, input_batch.reshape(_B * _L))
    return out.reshape(_B, 1)


# speedup vs baseline: 1.0118x; 1.0118x over previous
import functools

import jax
import jax.numpy as jnp
from jax import lax
from jax.experimental import pallas as pl
from jax.experimental.pallas import tpu as pltpu
from jax.experimental.pallas import tpu_sc as plsc

_VOCAB = 1000000
_EMB = 64
_HID = 128
_B = 16384
_L = 50

_NC, _NS, _LANES = 2, 16, 16
_NW = _NC * _NS

_CUT = 400000
_VH = _VOCAB - _CUT


def _fold_body(w1_ref, b1_ref, w2_ref, b2_ref, wc_ref):
    w2 = w2_ref[...]
    w = jnp.dot(w2, w1_ref[...], preferred_element_type=jnp.float32,
                precision=lax.Precision.HIGHEST)
    c = jnp.sum(w2 * b1_ref[...]) + b2_ref[0, 0]
    wc_ref[:, :_EMB] = w
    wc_ref[:, _EMB:] = jnp.full((1, 16), c, jnp.float32)


_fold = pl.pallas_call(
    _fold_body,
    out_shape=jax.ShapeDtypeStruct((1, _EMB + 16), jnp.float32),
)


_ROWS_BLK = 10000
_NSLICE = 4
_NBLK = _CUT // (_ROWS_BLK * _NSLICE)


def _score_tc_body(w1_ref, b1_ref, w2_ref, b2_ref, *refs):
    table_refs, s_ref = refs[:_NSLICE], refs[_NSLICE]
    w2 = w2_ref[...]
    w = jnp.dot(w2, w1_ref[...], preferred_element_type=jnp.float32,
                precision=lax.Precision.HIGHEST)
    c = jnp.sum(w2 * b1_ref[...]) + b2_ref[0, 0]
    for k in range(_NSLICE):
        t = table_refs[k][...]
        s = lax.dot_general(w, t, (((1,), (1,)), ((), ())),
                            preferred_element_type=jnp.float32)
        s_ref[:, k:k + 1, :] = ((s + c) * (1.0 / _L)).reshape(1, 1, _ROWS_BLK)


def _table_spec(k):
    return pl.BlockSpec((_ROWS_BLK, _EMB), lambda i, k=k: (i * _NSLICE + k, 0))


_score_tc = pl.pallas_call(
    _score_tc_body,
    grid=(_NBLK,),
    in_specs=[
        pl.BlockSpec((_HID, _EMB), lambda i: (0, 0)),
        pl.BlockSpec((1, _HID), lambda i: (0, 0)),
        pl.BlockSpec((1, _HID), lambda i: (0, 0)),
        pl.BlockSpec((1, 1), lambda i: (0, 0)),
    ] + [_table_spec(k) for k in range(_NSLICE)],
    out_specs=pl.BlockSpec((1, _NSLICE, _ROWS_BLK), lambda i: (i, 0, 0)),
    out_shape=jax.ShapeDtypeStruct((_NBLK, _NSLICE, _ROWS_BLK), jnp.float32),
)


_CR = 400
_RU = 8
_NCHT = _VH // _CR
_JMAX = 48
_NP = _EMB // _LANES

_sc_mesh = plsc.VectorSubcoreMesh(core_axis_name="c", subcore_axis_name="s")


@functools.partial(
    pl.kernel,
    mesh=_sc_mesh,
    out_type=jax.ShapeDtypeStruct((_VH,), jnp.float32),
    compiler_params=pltpu.CompilerParams(needs_layout_passes=False),
    scratch_types=[
        pltpu.VMEM((2, _CR, _EMB), jnp.float32),
        pltpu.VMEM((1, _EMB + 16), jnp.float32),
        pltpu.VMEM((_CR,), jnp.float32),
        pltpu.SemaphoreType.DMA,
        pltpu.SemaphoreType.DMA,
    ],
)
def _score_sc(table_hbm, wc_hbm, s_hbm, rows_v, wc_v, sv, sem0, sem1):
    wid = lax.axis_index("s") * _NC + lax.axis_index("c")
    pltpu.sync_copy(wc_hbm, wc_v)
    w_p = [wc_v[0, pl.ds(p * _LANES, _LANES)] for p in range(_NP)]
    cvec = wc_v[0, pl.ds(_EMB, _LANES)]
    lane = jnp.arange(_LANES, dtype=jnp.int32)
    sems = (sem0, sem1)

    def fire(j, buf):
        cid = wid + j * _NW

        @pl.when(cid < _NCHT)
        def _():
            pltpu.async_copy(
                table_hbm.at[pl.ds(cid * _CR, _CR), :], rows_v.at[buf],
                sems[buf])

    def process(j, buf):
        cid = wid + j * _NW

        @pl.when(cid < _NCHT)
        def _():
            pltpu.make_async_copy(
                table_hbm.at[pl.ds(cid * _CR, _CR), :], rows_v.at[buf],
                sems[buf]).wait()

            def row_body(g, carry):
                base = g * _RU
                for u in range(_RU):
                    r = base + u
                    part = rows_v[buf, r, pl.ds(0, _LANES)] * w_p[0]
                    for p in range(1, _NP):
                        part = part + rows_v[buf, r,
                                             pl.ds(p * _LANES,
                                                   _LANES)] * w_p[p]
                    tot = jnp.sum(part)
                    plsc.store_scatter(
                        sv,
                        [jnp.full((_LANES,), r, jnp.int32)],
                        jnp.full((_LANES,), tot, jnp.float32),
                        mask=lane == 0,
                    )
                return carry

            lax.fori_loop(0, _CR // _RU, row_body, 0)

            def post(g, carry):
                sl = pl.ds(g * _LANES, _LANES)
                sv[sl] = (sv[sl] + cvec) * (1.0 / _L)
                return carry

            lax.fori_loop(0, _CR // _LANES, post, 0)
            pltpu.sync_copy(sv, s_hbm.at[pl.ds(cid * _CR, _CR)])

    fire(0, 0)

    def outer(i2, carry):
        j0 = i2 * 2
        fire(j0 + 1, 1)
        process(j0, 0)
        fire(j0 + 2, 0)
        process(j0 + 1, 1)
        return carry

    lax.fori_loop(0, _JMAX // 2, outer, 0)


_TOK_W = (_B * _L) // _NW
_BAG_W = _B // _NW
_GRP = _BAG_W // _LANES


@functools.partial(
    pl.kernel,
    mesh=_sc_mesh,
    out_type=jax.ShapeDtypeStruct((_B,), jnp.float32),
    compiler_params=pltpu.CompilerParams(needs_layout_passes=False),
    scratch_types=[
        pltpu.VMEM((_TOK_W,), jnp.int32),
        pltpu.VMEM((_TOK_W,), jnp.float32),
        pltpu.VMEM((_BAG_W,), jnp.float32),
        pltpu.SemaphoreType.DMA,
    ],
)
def _pool(s_hbm, idx_hbm, out_hbm, idx_v, val_v, out_v, sem):
    wid = lax.axis_index("s") * _NC + lax.axis_index("c")
    tbase = wid * _TOK_W
    pltpu.sync_copy(idx_hbm.at[pl.ds(tbase, _TOK_W)], idx_v)
    pltpu.async_copy(s_hbm.at[idx_v], val_v, sem).wait()
    lane50 = jnp.arange(_LANES, dtype=jnp.int32) * _L

    def body(g, carry):
        base = lane50 + g * (_LANES * _L)
        acc = plsc.load_gather(val_v, [base])
        for l in range(1, _L):
            acc = acc + plsc.load_gather(val_v, [base + l])
        out_v[pl.ds(g * _LANES, _LANES)] = acc
        return carry

    lax.fori_loop(0, _GRP, body, 0)
    pltpu.sync_copy(out_v, out_hbm.at[pl.ds(wid * _BAG_W, _BAG_W)])


def kernel(input_batch, emb_table, W1, b1, W2, b2):
    b1r = b1.reshape(1, _HID)
    b2r = b2.reshape(1, 1)
    wc = _fold(W1, b1r, W2, b2r)
    table_hi = lax.slice(emb_table, (_CUT, 0), (_VOCAB, _EMB))
    s_hi = _score_sc(table_hi, wc)
    s_lo = _score_tc(W1, b1r, W2, b2r, *([emb_table] * _NSLICE))
    s = jnp.concatenate([s_lo.reshape(_CUT), s_hi])
    out = _pool(s, input_batch.reshape(_B * _L))
    return out.reshape(_B, 1)


# speedup vs baseline: 6.2418x; 6.1691x over previous
import functools

import jax
import jax.numpy as jnp
from jax import lax
from jax.experimental import pallas as pl
from jax.experimental.pallas import tpu as pltpu
from jax.experimental.pallas import tpu_sc as plsc

_VOCAB = 1000000
_EMB = 64
_HID = 128
_B = 16384
_L = 50


_BLKC = 51200
_NBLK = -(-_VOCAB // _BLKC)


def _score_body(w1_ref, b1_ref, w2_ref, b2_ref, tt_ref, s_ref):
    w2 = w2_ref[...]
    w = jnp.dot(w2, w1_ref[...], preferred_element_type=jnp.float32,
                precision=lax.Precision.HIGHEST)
    c = jnp.sum(w2 * b1_ref[...]) + b2_ref[0, 0]
    t = tt_ref[...]
    s = lax.dot_general(w, t, (((1,), (0,)), ((), ())),
                        preferred_element_type=jnp.float32)
    s_ref[...] = ((s + c) * (1.0 / _L)).reshape(1, 1, _BLKC)


_score = pl.pallas_call(
    _score_body,
    grid=(_NBLK,),
    in_specs=[
        pl.BlockSpec((_HID, _EMB), lambda i: (0, 0)),
        pl.BlockSpec((1, _HID), lambda i: (0, 0)),
        pl.BlockSpec((1, _HID), lambda i: (0, 0)),
        pl.BlockSpec((1, 1), lambda i: (0, 0)),
        pl.BlockSpec((_EMB, _BLKC), lambda i: (0, i)),
    ],
    out_specs=pl.BlockSpec((1, 1, _BLKC), lambda i: (i, 0, 0)),
    out_shape=jax.ShapeDtypeStruct((_NBLK, 1, _BLKC), jnp.float32),
)


_NC, _NS, _LANES = 2, 16, 16
_NW = _NC * _NS
_TOK_W = (_B * _L) // _NW
_BAG_W = _B // _NW
_GRP = _BAG_W // _LANES

_sc_mesh = plsc.VectorSubcoreMesh(core_axis_name="c", subcore_axis_name="s")


@functools.partial(
    pl.kernel,
    mesh=_sc_mesh,
    out_type=jax.ShapeDtypeStruct((_B,), jnp.float32),
    compiler_params=pltpu.CompilerParams(needs_layout_passes=False),
    scratch_types=[
        pltpu.VMEM((_TOK_W,), jnp.int32),
        pltpu.VMEM((_TOK_W,), jnp.float32),
        pltpu.VMEM((_BAG_W,), jnp.float32),
        pltpu.SemaphoreType.DMA,
    ],
)
def _pool(s_hbm, idx_hbm, out_hbm, idx_v, val_v, out_v, sem):
    wid = lax.axis_index("s") * _NC + lax.axis_index("c")
    tbase = wid * _TOK_W
    pltpu.sync_copy(idx_hbm.at[pl.ds(tbase, _TOK_W)], idx_v)
    pltpu.async_copy(s_hbm.at[idx_v], val_v, sem).wait()
    lane50 = jnp.arange(_LANES, dtype=jnp.int32) * _L

    def body(g, carry):
        base = lane50 + g * (_LANES * _L)
        acc = plsc.load_gather(val_v, [base])
        for l in range(1, _L):
            acc = acc + plsc.load_gather(val_v, [base + l])
        out_v[pl.ds(g * _LANES, _LANES)] = acc
        return carry

    lax.fori_loop(0, _GRP, body, 0)
    pltpu.sync_copy(out_v, out_hbm.at[pl.ds(wid * _BAG_W, _BAG_W)])


def kernel(input_batch, emb_table, W1, b1, W2, b2):
    s = _score(W1, b1.reshape(1, _HID), W2, b2.reshape(1, 1), emb_table.T)
    out = _pool(s.reshape(_NBLK * _BLKC), input_batch.reshape(_B * _L))
    return out.reshape(_B, 1)
